# Initial kernel scaffold; baseline (speedup 1.0000x reference)
import jax
import jax.numpy as jnp
from jax.experimental import pallas as pl


def kernel(feature, edge_index, W, b):
    raise NotImplementedError("write your pallas kernel here")


# speedup vs baseline: 5.0505x; 5.0505x over previous
import functools

import jax
import jax.numpy as jnp
from jax import lax
from jax.experimental import pallas as pl
from jax.experimental.pallas import tpu as pltpu
from jax.experimental.pallas import tpu_sc as plsc

N = 10000
E = 320000
D = 128

NC = 2
NS = 16
NW = NC * NS
EPW = E // NW
K = 80
NCHUNK = EPW // K
AP = 624
TAIL = 16
ZB = 208

_mesh = plsc.VectorSubcoreMesh(core_axis_name="c", subcore_axis_name="s")


def _sc_body(with_deg, *refs):
    if with_deg:
        (h_hbm, src_hbm, dst_hbm, agg_out, deg_out,
         sidx, didx, rows, onesb, zrow, zdeg, agg_sh, deg_sh, sem) = refs
    else:
        (h_hbm, src_hbm, dst_hbm, agg_out,
         sidx, didx, rows, zrow, agg_sh, sem) = refs

    c = lax.axis_index("c")
    s = lax.axis_index("s")
    w = c * NS + s
    zeros16 = jnp.zeros((16,), jnp.float32)

    def _zr(i, carry):
        for k8 in range(D // 16):
            zrow[i, pl.ds(k8 * 16, 16)] = zeros16
        return carry
    lax.fori_loop(0, ZB, _zr, 0)
    base = s * AP
    for j in range(AP // ZB):
        pltpu.sync_copy(zrow, agg_sh.at[pl.ds(base + j * ZB, ZB)])

    @pl.when(s == NS - 1)
    def _():
        pltpu.sync_copy(zrow.at[pl.ds(0, TAIL)], agg_sh.at[pl.ds(N - TAIL, TAIL)])

    if with_deg:
        def _zd(i, carry):
            zdeg[pl.ds(i * 16, 16)] = zeros16
            return carry
        lax.fori_loop(0, ZB // 16, _zd, 0)
        for j in range(AP // ZB):
            pltpu.sync_copy(zdeg, deg_sh.at[pl.ds(base + j * ZB, ZB)])

        @pl.when(s == NS - 1)
        def _():
            pltpu.sync_copy(zdeg.at[pl.ds(0, TAIL)],
                            deg_sh.at[pl.ds(N - TAIL, TAIL)])

        ones16 = jnp.ones((16,), jnp.float32)
        def _on(i, carry):
            onesb[pl.ds(i * 16, 16)] = ones16
            return carry
        lax.fori_loop(0, K // 16, _on, 0)

    plsc.subcore_barrier()

    ebase = w * EPW
    def _chunk(g, carry):
        off = ebase + g * K
        pltpu.sync_copy(src_hbm.at[pl.ds(off, K)], sidx)
        pltpu.sync_copy(dst_hbm.at[pl.ds(off, K)], didx)
        pltpu.async_copy(h_hbm.at[sidx], rows, sem).wait()
        pltpu.sync_copy(rows, agg_sh.at[didx], add=True)
        if with_deg:
            pltpu.sync_copy(onesb, deg_sh.at[didx], add=True)
        return carry
    lax.fori_loop(0, NCHUNK, _chunk, 0)

    plsc.subcore_barrier()

    pltpu.sync_copy(agg_sh.at[pl.ds(base, AP)], agg_out.at[c, pl.ds(base, AP)])

    @pl.when(s == NS - 1)
    def _():
        pltpu.sync_copy(agg_sh.at[pl.ds(N - TAIL, TAIL)],
                        agg_out.at[c, pl.ds(N - TAIL, TAIL)])

    if with_deg:
        for j in range(AP // ZB):
            pltpu.sync_copy(deg_sh.at[pl.ds(base + j * ZB, ZB)], zdeg)
            pltpu.sync_copy(zdeg, deg_out.at[pl.ds(c * N + base + j * ZB, ZB)])

        @pl.when(s == NS - 1)
        def _():
            pltpu.sync_copy(deg_sh.at[pl.ds(N - TAIL, TAIL)],
                            zdeg.at[pl.ds(0, TAIL)])
            pltpu.sync_copy(zdeg.at[pl.ds(0, TAIL)],
                            deg_out.at[pl.ds(c * N + N - TAIL, TAIL)])


_round1 = pl.kernel(
    functools.partial(_sc_body, True),
    out_type=(jax.ShapeDtypeStruct((NC, N, D), jnp.float32),
              jax.ShapeDtypeStruct((NC * N,), jnp.float32)),
    mesh=_mesh,
    scratch_types=[
        pltpu.VMEM((K,), jnp.int32),
        pltpu.VMEM((K,), jnp.int32),
        pltpu.VMEM((K, D), jnp.float32),
        pltpu.VMEM((K,), jnp.float32),
        pltpu.VMEM((ZB, D), jnp.float32),
        pltpu.VMEM((ZB,), jnp.float32),
        pltpu.VMEM_SHARED((N, D), jnp.float32),
        pltpu.VMEM_SHARED((N,), jnp.float32),
        pltpu.SemaphoreType.DMA,
    ],
)

_round2 = pl.kernel(
    functools.partial(_sc_body, False),
    out_type=jax.ShapeDtypeStruct((NC, N, D), jnp.float32),
    mesh=_mesh,
    scratch_types=[
        pltpu.VMEM((K,), jnp.int32),
        pltpu.VMEM((K,), jnp.int32),
        pltpu.VMEM((K, D), jnp.float32),
        pltpu.VMEM((ZB, D), jnp.float32),
        pltpu.VMEM_SHARED((N, D), jnp.float32),
        pltpu.SemaphoreType.DMA,
    ],
)


_RB = 1000


def _combine_body(agg_ref, deg_ref, h_ref, out_ref):
    msk = ((deg_ref[:, 0] + deg_ref[:, 1]) > 0.0)[:, None]
    agg = agg_ref[0] + agg_ref[1]
    out_ref[...] = jnp.where(msk, agg, h_ref[...])


def _final_body(agg_ref, deg_ref, h_ref, w_ref, b_ref, out_ref):
    msk = ((deg_ref[:, 0] + deg_ref[:, 1]) > 0.0)[:, None]
    h2 = jnp.where(msk, agg_ref[0] + agg_ref[1], h_ref[...])
    acc = lax.dot_general(h2, w_ref[...], (((1,), (1,)), ((), ())),
                          preferred_element_type=jnp.float32,
                          precision=lax.Precision.HIGHEST)
    out_ref[...] = acc + b_ref[...]


_combine = pl.pallas_call(
    _combine_body,
    grid=(N // _RB,),
    in_specs=[
        pl.BlockSpec((NC, _RB, D), lambda i: (0, i, 0)),
        pl.BlockSpec((_RB, NC), lambda i: (i, 0)),
        pl.BlockSpec((_RB, D), lambda i: (i, 0)),
    ],
    out_specs=pl.BlockSpec((_RB, D), lambda i: (i, 0)),
    out_shape=jax.ShapeDtypeStruct((N, D), jnp.float32),
)

_final = pl.pallas_call(
    _final_body,
    grid=(N // _RB,),
    in_specs=[
        pl.BlockSpec((NC, _RB, D), lambda i: (0, i, 0)),
        pl.BlockSpec((_RB, NC), lambda i: (i, 0)),
        pl.BlockSpec((_RB, D), lambda i: (i, 0)),
        pl.BlockSpec((D, D), lambda i: (0, 0)),
        pl.BlockSpec((1, D), lambda i: (0, 0)),
    ],
    out_specs=pl.BlockSpec((_RB, D), lambda i: (i, 0)),
    out_shape=jax.ShapeDtypeStruct((N, D), jnp.float32),
)


def kernel(feature, edge_index, W, b):
    src = edge_index[0]
    dst = edge_index[1]
    agg1, degp = _round1(feature, src, dst)
    deg2 = degp.reshape(NC, N).T
    h1 = _combine(agg1, deg2, feature)
    agg2 = _round2(h1, src, dst)
    return _final(agg2, deg2, h1, W, b.reshape(1, D))


# speedup vs baseline: 9.2888x; 1.8392x over previous
import functools

import jax
import jax.numpy as jnp
from jax import lax
from jax.experimental import pallas as pl
from jax.experimental.pallas import tpu as pltpu
from jax.experimental.pallas import tpu_sc as plsc

N = 10000
E = 320000
D = 128

NC = 2
NS = 16
NW = NC * NS
EPW = E // NW
K = 80
NCHUNK = EPW // K
AP = 624
TAIL = 16
ZB = 48

_mesh = plsc.VectorSubcoreMesh(core_axis_name="c", subcore_axis_name="s")


NBUF = 3


def _sc_body(with_deg, *refs):
    if with_deg:
        (h_hbm, src_hbm, dst_hbm, agg_out, deg_out) = refs[:5]
        refs = refs[5:]
        sidx = refs[0:NBUF]; didx = refs[NBUF:2 * NBUF]; rows = refs[2 * NBUF:3 * NBUF]
        (onesb, zrow, zdeg, agg_sh, deg_sh) = refs[3 * NBUF:3 * NBUF + 5]
        sems = refs[3 * NBUF + 5:]
        semi = sems[0:NBUF]; semg = sems[NBUF:2 * NBUF]; sems_ = sems[2 * NBUF:3 * NBUF]
    else:
        (h_hbm, src_hbm, dst_hbm, agg_out) = refs[:4]
        refs = refs[4:]
        sidx = refs[0:NBUF]; didx = refs[NBUF:2 * NBUF]; rows = refs[2 * NBUF:3 * NBUF]
        (zrow, agg_sh) = refs[3 * NBUF:3 * NBUF + 2]
        sems = refs[3 * NBUF + 2:]
        semi = sems[0:NBUF]; semg = sems[NBUF:2 * NBUF]; sems_ = sems[2 * NBUF:3 * NBUF]

    c = lax.axis_index("c")
    s = lax.axis_index("s")
    w = c * NS + s
    zeros16 = jnp.zeros((16,), jnp.float32)

    def _zr(i, carry):
        for k8 in range(D // 16):
            zrow[i, pl.ds(k8 * 16, 16)] = zeros16
        return carry
    lax.fori_loop(0, ZB, _zr, 0)
    base = s * AP
    for j in range(AP // ZB):
        pltpu.sync_copy(zrow, agg_sh.at[pl.ds(base + j * ZB, ZB)])

    @pl.when(s == NS - 1)
    def _():
        pltpu.sync_copy(zrow.at[pl.ds(0, TAIL)], agg_sh.at[pl.ds(N - TAIL, TAIL)])

    if with_deg:
        def _zd(i, carry):
            zdeg[pl.ds(i * 16, 16)] = zeros16
            return carry
        lax.fori_loop(0, ZB // 16, _zd, 0)
        for j in range(AP // ZB):
            pltpu.sync_copy(zdeg, deg_sh.at[pl.ds(base + j * ZB, ZB)])

        @pl.when(s == NS - 1)
        def _():
            pltpu.sync_copy(zdeg.at[pl.ds(0, TAIL)],
                            deg_sh.at[pl.ds(N - TAIL, TAIL)])

        ones16 = jnp.ones((16,), jnp.float32)
        def _on(i, carry):
            onesb[pl.ds(i * 16, 16)] = ones16
            return carry
        lax.fori_loop(0, K // 16, _on, 0)

    plsc.subcore_barrier()

    ebase = w * EPW

    def start_idx(g, b):
        off = ebase + g * K
        pltpu.async_copy(src_hbm.at[pl.ds(off, K)], sidx[b], semi[b])
        pltpu.async_copy(dst_hbm.at[pl.ds(off, K)], didx[b], semi[b])

    def wait_idx(g, b):
        off = ebase + g * K
        pltpu.make_async_copy(src_hbm.at[pl.ds(off, K)], sidx[b], semi[b]).wait()
        pltpu.make_async_copy(dst_hbm.at[pl.ds(off, K)], didx[b], semi[b]).wait()

    def start_gather(b):
        pltpu.async_copy(h_hbm.at[sidx[b]], rows[b], semg[b])

    def wait_gather(b):
        pltpu.make_async_copy(h_hbm.at[sidx[b]], rows[b], semg[b]).wait()

    def start_scatter(b):
        pltpu.async_copy(rows[b], agg_sh.at[didx[b]], sems_[b], add=True)
        if with_deg:
            pltpu.async_copy(onesb, deg_sh.at[didx[b]], sems_[b], add=True)

    def wait_scatter(b):
        pltpu.make_async_copy(rows[b], agg_sh.at[didx[b]], sems_[b]).wait()
        if with_deg:
            pltpu.make_async_copy(onesb, deg_sh.at[didx[b]], sems_[b]).wait()

    def step(g, b, lo, hi):
        wait_gather(b)
        start_scatter(b)
        if lo - 1 >= 0:
            wait_scatter((b - 1) % NBUF)
        if hi + 2 <= NCHUNK:
            start_idx(g + 2, (b + 2) % NBUF)
        if hi + 1 <= NCHUNK:
            wait_idx(g + 1, (b + 1) % NBUF)
            start_gather((b + 1) % NBUF)

    for g in range(2):
        start_idx(g, g)
    wait_idx(0, 0)
    start_gather(0)
    for g in range(NBUF):
        step(g, g, g, g + 1)

    def _outer(g2, carry):
        for j in range(NBUF):
            step(g2 * NBUF + j, j, NBUF, NCHUNK - 2)
        return carry
    lax.fori_loop(1, (NCHUNK - 2) // NBUF, _outer, 0)

    for g in range(NCHUNK - 2, NCHUNK):
        step(g, g % NBUF, g, g + 1)
    wait_scatter((NCHUNK - 1) % NBUF)

    plsc.subcore_barrier()

    pltpu.sync_copy(agg_sh.at[pl.ds(base, AP)], agg_out.at[c, pl.ds(base, AP)])

    @pl.when(s == NS - 1)
    def _():
        pltpu.sync_copy(agg_sh.at[pl.ds(N - TAIL, TAIL)],
                        agg_out.at[c, pl.ds(N - TAIL, TAIL)])

    if with_deg:
        for j in range(AP // ZB):
            pltpu.sync_copy(deg_sh.at[pl.ds(base + j * ZB, ZB)], zdeg)
            pltpu.sync_copy(zdeg, deg_out.at[pl.ds(c * N + base + j * ZB, ZB)])

        @pl.when(s == NS - 1)
        def _():
            pltpu.sync_copy(deg_sh.at[pl.ds(N - TAIL, TAIL)],
                            zdeg.at[pl.ds(0, TAIL)])
            pltpu.sync_copy(zdeg.at[pl.ds(0, TAIL)],
                            deg_out.at[pl.ds(c * N + N - TAIL, TAIL)])


_round1 = pl.kernel(
    functools.partial(_sc_body, True),
    out_type=(jax.ShapeDtypeStruct((NC, N, D), jnp.float32),
              jax.ShapeDtypeStruct((NC * N,), jnp.float32)),
    mesh=_mesh,
    scratch_types=(
        [pltpu.VMEM((K,), jnp.int32)] * NBUF
        + [pltpu.VMEM((K,), jnp.int32)] * NBUF
        + [pltpu.VMEM((K, D), jnp.float32)] * NBUF
        + [
            pltpu.VMEM((K,), jnp.float32),
            pltpu.VMEM((ZB, D), jnp.float32),
            pltpu.VMEM((ZB,), jnp.float32),
            pltpu.VMEM_SHARED((N, D), jnp.float32),
            pltpu.VMEM_SHARED((N,), jnp.float32),
        ]
        + [pltpu.SemaphoreType.DMA] * (3 * NBUF)
    ),
)

_round2 = pl.kernel(
    functools.partial(_sc_body, False),
    out_type=jax.ShapeDtypeStruct((NC, N, D), jnp.float32),
    mesh=_mesh,
    scratch_types=(
        [pltpu.VMEM((K,), jnp.int32)] * NBUF
        + [pltpu.VMEM((K,), jnp.int32)] * NBUF
        + [pltpu.VMEM((K, D), jnp.float32)] * NBUF
        + [
            pltpu.VMEM((ZB, D), jnp.float32),
            pltpu.VMEM_SHARED((N, D), jnp.float32),
        ]
        + [pltpu.SemaphoreType.DMA] * (3 * NBUF)
    ),
)


_RB = 1000


def _combine_body(agg_ref, deg_ref, h_ref, out_ref):
    msk = ((deg_ref[:, 0] + deg_ref[:, 1]) > 0.0)[:, None]
    agg = agg_ref[0] + agg_ref[1]
    out_ref[...] = jnp.where(msk, agg, h_ref[...])


def _final_body(agg_ref, deg_ref, h_ref, w_ref, b_ref, out_ref):
    msk = ((deg_ref[:, 0] + deg_ref[:, 1]) > 0.0)[:, None]
    h2 = jnp.where(msk, agg_ref[0] + agg_ref[1], h_ref[...])
    acc = lax.dot_general(h2, w_ref[...], (((1,), (1,)), ((), ())),
                          preferred_element_type=jnp.float32,
                          precision=lax.Precision.HIGHEST)
    out_ref[...] = acc + b_ref[...]


_combine = pl.pallas_call(
    _combine_body,
    grid=(N // _RB,),
    in_specs=[
        pl.BlockSpec((NC, _RB, D), lambda i: (0, i, 0)),
        pl.BlockSpec((_RB, NC), lambda i: (i, 0)),
        pl.BlockSpec((_RB, D), lambda i: (i, 0)),
    ],
    out_specs=pl.BlockSpec((_RB, D), lambda i: (i, 0)),
    out_shape=jax.ShapeDtypeStruct((N, D), jnp.float32),
)

_final = pl.pallas_call(
    _final_body,
    grid=(N // _RB,),
    in_specs=[
        pl.BlockSpec((NC, _RB, D), lambda i: (0, i, 0)),
        pl.BlockSpec((_RB, NC), lambda i: (i, 0)),
        pl.BlockSpec((_RB, D), lambda i: (i, 0)),
        pl.BlockSpec((D, D), lambda i: (0, 0)),
        pl.BlockSpec((1, D), lambda i: (0, 0)),
    ],
    out_specs=pl.BlockSpec((_RB, D), lambda i: (i, 0)),
    out_shape=jax.ShapeDtypeStruct((N, D), jnp.float32),
)


def kernel(feature, edge_index, W, b):
    src = edge_index[0]
    dst = edge_index[1]
    agg1, degp = _round1(feature, src, dst)
    deg2 = degp.reshape(NC, N).T
    h1 = _combine(agg1, deg2, feature)
    agg2 = _round2(h1, src, dst)
    return _final(agg2, deg2, h1, W, b.reshape(1, D))


# speedup vs baseline: 13.2517x; 1.4266x over previous
import functools

import jax
import jax.numpy as jnp
from jax import lax
from jax.experimental import pallas as pl
from jax.experimental.pallas import tpu as pltpu
from jax.experimental.pallas import tpu_sc as plsc

N = 10000
E = 320000
D = 128

NC = 2
NS = 16
NW = NC * NS
EPW = E // NW
K = 80
NCHUNK = EPW // K
AP = 624
TAIL = 16
ZB = 48

_mesh = plsc.VectorSubcoreMesh(core_axis_name="c", subcore_axis_name="s")


NBUF = 4


def _sc_body(with_deg, *refs):
    if with_deg:
        (h_hbm, src_hbm, dst_hbm, agg_out, deg_out) = refs[:5]
        refs = refs[5:]
        sidx = refs[0:NBUF]; didx = refs[NBUF:2 * NBUF]; rows = refs[2 * NBUF:3 * NBUF]
        (onesb, zrow, zdeg, agg_sh, deg_sh) = refs[3 * NBUF:3 * NBUF + 5]
        sems = refs[3 * NBUF + 5:]
        semi = sems[0:NBUF]; semg = sems[NBUF:2 * NBUF]; sems_ = sems[2 * NBUF:3 * NBUF]
    else:
        (h_hbm, src_hbm, dst_hbm, agg_out) = refs[:4]
        refs = refs[4:]
        sidx = refs[0:NBUF]; didx = refs[NBUF:2 * NBUF]; rows = refs[2 * NBUF:3 * NBUF]
        (zrow, agg_sh) = refs[3 * NBUF:3 * NBUF + 2]
        sems = refs[3 * NBUF + 2:]
        semi = sems[0:NBUF]; semg = sems[NBUF:2 * NBUF]; sems_ = sems[2 * NBUF:3 * NBUF]

    c = lax.axis_index("c")
    s = lax.axis_index("s")
    w = c * NS + s
    zeros16 = jnp.zeros((16,), jnp.float32)

    def _zr(i, carry):
        for k8 in range(D // 16):
            zrow[i, pl.ds(k8 * 16, 16)] = zeros16
        return carry
    lax.fori_loop(0, ZB, _zr, 0)
    base = s * AP
    for j in range(AP // ZB):
        pltpu.sync_copy(zrow, agg_sh.at[pl.ds(base + j * ZB, ZB)])

    @pl.when(s == NS - 1)
    def _():
        pltpu.sync_copy(zrow.at[pl.ds(0, TAIL)], agg_sh.at[pl.ds(N - TAIL, TAIL)])

    if with_deg:
        def _zd(i, carry):
            zdeg[pl.ds(i * 16, 16)] = zeros16
            return carry
        lax.fori_loop(0, ZB // 16, _zd, 0)
        for j in range(AP // ZB):
            pltpu.sync_copy(zdeg, deg_sh.at[pl.ds(base + j * ZB, ZB)])

        @pl.when(s == NS - 1)
        def _():
            pltpu.sync_copy(zdeg.at[pl.ds(0, TAIL)],
                            deg_sh.at[pl.ds(N - TAIL, TAIL)])

        ones16 = jnp.ones((16,), jnp.float32)
        def _on(i, carry):
            onesb[pl.ds(i * 16, 16)] = ones16
            return carry
        lax.fori_loop(0, K // 16, _on, 0)

    plsc.subcore_barrier()

    ebase = w * EPW

    def start_idx(g, b):
        off = ebase + g * K
        pltpu.async_copy(src_hbm.at[pl.ds(off, K)], sidx[b], semi[b])
        pltpu.async_copy(dst_hbm.at[pl.ds(off, K)], didx[b], semi[b])

    def wait_idx(g, b):
        off = ebase + g * K
        pltpu.make_async_copy(src_hbm.at[pl.ds(off, K)], sidx[b], semi[b]).wait()
        pltpu.make_async_copy(dst_hbm.at[pl.ds(off, K)], didx[b], semi[b]).wait()

    def start_gather(b):
        pltpu.async_copy(h_hbm.at[sidx[b]], rows[b], semg[b])

    def wait_gather(b):
        pltpu.make_async_copy(h_hbm.at[sidx[b]], rows[b], semg[b]).wait()

    def start_scatter(b):
        pltpu.async_copy(rows[b], agg_sh.at[didx[b]], sems_[b], add=True)
        if with_deg:
            pltpu.async_copy(onesb, deg_sh.at[didx[b]], sems_[b], add=True)

    def wait_scatter(b):
        pltpu.make_async_copy(rows[b], agg_sh.at[didx[b]], sems_[b]).wait()
        if with_deg:
            pltpu.make_async_copy(onesb, deg_sh.at[didx[b]], sems_[b]).wait()

    def step(g, b, lo, hi):
        wait_gather(b)
        start_scatter(b)
        if lo - 1 >= 0:
            wait_scatter((b + NBUF - 1) % NBUF)
        if hi + 3 <= NCHUNK:
            start_idx(g + 3, (b + 3) % NBUF)
        if hi + 2 <= NCHUNK:
            wait_idx(g + 2, (b + 2) % NBUF)
            start_gather((b + 2) % NBUF)

    for g in range(3):
        start_idx(g, g)
    for g in range(2):
        wait_idx(g, g)
        start_gather(g)
    for g in range(NBUF):
        step(g, g, g, g + 1)

    def _outer(g2, carry):
        for j in range(NBUF):
            step(g2 * NBUF + j, j, NBUF, NCHUNK - 5)
        return carry
    lax.fori_loop(1, (NCHUNK - 5) // NBUF, _outer, 0)

    for g in range(NCHUNK - 5, NCHUNK):
        step(g, g % NBUF, g, g + 1)
    wait_scatter((NCHUNK - 1) % NBUF)

    plsc.subcore_barrier()

    pltpu.sync_copy(agg_sh.at[pl.ds(base, AP)], agg_out.at[c, pl.ds(base, AP)])

    @pl.when(s == NS - 1)
    def _():
        pltpu.sync_copy(agg_sh.at[pl.ds(N - TAIL, TAIL)],
                        agg_out.at[c, pl.ds(N - TAIL, TAIL)])

    if with_deg:
        for j in range(AP // ZB):
            pltpu.sync_copy(deg_sh.at[pl.ds(base + j * ZB, ZB)], zdeg)
            pltpu.sync_copy(zdeg, deg_out.at[pl.ds(c * N + base + j * ZB, ZB)])

        @pl.when(s == NS - 1)
        def _():
            pltpu.sync_copy(deg_sh.at[pl.ds(N - TAIL, TAIL)],
                            zdeg.at[pl.ds(0, TAIL)])
            pltpu.sync_copy(zdeg.at[pl.ds(0, TAIL)],
                            deg_out.at[pl.ds(c * N + N - TAIL, TAIL)])


_round1 = pl.kernel(
    functools.partial(_sc_body, True),
    out_type=(jax.ShapeDtypeStruct((NC, N, D), jnp.float32),
              jax.ShapeDtypeStruct((NC * N,), jnp.float32)),
    mesh=_mesh,
    scratch_types=(
        [pltpu.VMEM((K,), jnp.int32)] * NBUF
        + [pltpu.VMEM((K,), jnp.int32)] * NBUF
        + [pltpu.VMEM((K, D), jnp.float32)] * NBUF
        + [
            pltpu.VMEM((K,), jnp.float32),
            pltpu.VMEM((ZB, D), jnp.float32),
            pltpu.VMEM((ZB,), jnp.float32),
            pltpu.VMEM_SHARED((N, D), jnp.float32),
            pltpu.VMEM_SHARED((N,), jnp.float32),
        ]
        + [pltpu.SemaphoreType.DMA] * (3 * NBUF)
    ),
)

_round2 = pl.kernel(
    functools.partial(_sc_body, False),
    out_type=jax.ShapeDtypeStruct((NC, N, D), jnp.float32),
    mesh=_mesh,
    scratch_types=(
        [pltpu.VMEM((K,), jnp.int32)] * NBUF
        + [pltpu.VMEM((K,), jnp.int32)] * NBUF
        + [pltpu.VMEM((K, D), jnp.float32)] * NBUF
        + [
            pltpu.VMEM((ZB, D), jnp.float32),
            pltpu.VMEM_SHARED((N, D), jnp.float32),
        ]
        + [pltpu.SemaphoreType.DMA] * (3 * NBUF)
    ),
)


_RB = 1000


def _combine_body(agg_ref, deg_ref, h_ref, out_ref):
    msk = ((deg_ref[:, 0] + deg_ref[:, 1]) > 0.0)[:, None]
    agg = agg_ref[0] + agg_ref[1]
    out_ref[...] = jnp.where(msk, agg, h_ref[...])


def _final_body(agg_ref, deg_ref, h_ref, w_ref, b_ref, out_ref):
    msk = ((deg_ref[:, 0] + deg_ref[:, 1]) > 0.0)[:, None]
    h2 = jnp.where(msk, agg_ref[0] + agg_ref[1], h_ref[...])
    acc = lax.dot_general(h2, w_ref[...], (((1,), (1,)), ((), ())),
                          preferred_element_type=jnp.float32,
                          precision=lax.Precision.HIGHEST)
    out_ref[...] = acc + b_ref[...]


_combine = pl.pallas_call(
    _combine_body,
    grid=(N // _RB,),
    in_specs=[
        pl.BlockSpec((NC, _RB, D), lambda i: (0, i, 0)),
        pl.BlockSpec((_RB, NC), lambda i: (i, 0)),
        pl.BlockSpec((_RB, D), lambda i: (i, 0)),
    ],
    out_specs=pl.BlockSpec((_RB, D), lambda i: (i, 0)),
    out_shape=jax.ShapeDtypeStruct((N, D), jnp.float32),
)

_final = pl.pallas_call(
    _final_body,
    grid=(N // _RB,),
    in_specs=[
        pl.BlockSpec((NC, _RB, D), lambda i: (0, i, 0)),
        pl.BlockSpec((_RB, NC), lambda i: (i, 0)),
        pl.BlockSpec((_RB, D), lambda i: (i, 0)),
        pl.BlockSpec((D, D), lambda i: (0, 0)),
        pl.BlockSpec((1, D), lambda i: (0, 0)),
    ],
    out_specs=pl.BlockSpec((_RB, D), lambda i: (i, 0)),
    out_shape=jax.ShapeDtypeStruct((N, D), jnp.float32),
)


def kernel(feature, edge_index, W, b):
    src = edge_index[0]
    dst = edge_index[1]
    agg1, degp = _round1(feature, src, dst)
    deg2 = degp.reshape(NC, N).T
    h1 = _combine(agg1, deg2, feature)
    agg2 = _round2(h1, src, dst)
    return _final(agg2, deg2, h1, W, b.reshape(1, D))


# speedup vs baseline: 13.5291x; 1.0209x over previous
import functools

import jax
import jax.numpy as jnp
from jax import lax
from jax.experimental import pallas as pl
from jax.experimental.pallas import tpu as pltpu
from jax.experimental.pallas import tpu_sc as plsc

N = 10000
E = 320000
D = 128

NC = 2
NS = 16
NW = NC * NS
EPW = E // NW
K = 80
NCHUNK = EPW // K
AP = 624
TAIL = 16
ZB = 48

_mesh = plsc.VectorSubcoreMesh(core_axis_name="c", subcore_axis_name="s")


NBUF = 4
SD = 1
IL = 3
GL = 2


def _sc_body(with_deg, *refs):
    if with_deg:
        (h_hbm, src_hbm, dst_hbm, agg_out, deg_out) = refs[:5]
        refs = refs[5:]
        sidx = refs[0:NBUF]; didx = refs[NBUF:2 * NBUF]; rows = refs[2 * NBUF:3 * NBUF]
        (onesb, zrow, zdeg, agg_sh, deg_sh) = refs[3 * NBUF:3 * NBUF + 5]
        sems = refs[3 * NBUF + 5:]
        semi = sems[0:NBUF]; semg = sems[NBUF:2 * NBUF]; sems_ = sems[2 * NBUF:3 * NBUF]
        semz = sems[3 * NBUF]
    else:
        (h_hbm, src_hbm, dst_hbm, agg_out) = refs[:4]
        refs = refs[4:]
        sidx = refs[0:NBUF]; didx = refs[NBUF:2 * NBUF]; rows = refs[2 * NBUF:3 * NBUF]
        (zrow, agg_sh) = refs[3 * NBUF:3 * NBUF + 2]
        sems = refs[3 * NBUF + 2:]
        semi = sems[0:NBUF]; semg = sems[NBUF:2 * NBUF]; sems_ = sems[2 * NBUF:3 * NBUF]
        semz = sems[3 * NBUF]

    c = lax.axis_index("c")
    s = lax.axis_index("s")
    w = c * NS + s
    zeros16 = jnp.zeros((16,), jnp.float32)

    def _zr(i, carry):
        for k8 in range(D // 16):
            zrow[i, pl.ds(k8 * 16, 16)] = zeros16
        return carry
    lax.fori_loop(0, ZB, _zr, 0)
    base = s * AP
    for j in range(AP // ZB):
        pltpu.async_copy(zrow, agg_sh.at[pl.ds(base + j * ZB, ZB)], semz)

    @pl.when(s == NS - 1)
    def _():
        pltpu.async_copy(zrow.at[pl.ds(0, TAIL)],
                         agg_sh.at[pl.ds(N - TAIL, TAIL)], semz)

    if with_deg:
        def _zd(i, carry):
            zdeg[pl.ds(i * 16, 16)] = zeros16
            return carry
        lax.fori_loop(0, ZB // 16, _zd, 0)
        for j in range(AP // ZB):
            pltpu.async_copy(zdeg, deg_sh.at[pl.ds(base + j * ZB, ZB)], semz)

        @pl.when(s == NS - 1)
        def _():
            pltpu.async_copy(zdeg.at[pl.ds(0, TAIL)],
                             deg_sh.at[pl.ds(N - TAIL, TAIL)], semz)

        ones16 = jnp.ones((16,), jnp.float32)
        def _on(i, carry):
            onesb[pl.ds(i * 16, 16)] = ones16
            return carry
        lax.fori_loop(0, K // 16, _on, 0)

    def wait_zero():
        for j in range(AP // ZB):
            pltpu.make_async_copy(
                zrow, agg_sh.at[pl.ds(base + j * ZB, ZB)], semz).wait()
        if with_deg:
            for j in range(AP // ZB):
                pltpu.make_async_copy(
                    zdeg, deg_sh.at[pl.ds(base + j * ZB, ZB)], semz).wait()

        @pl.when(s == NS - 1)
        def _():
            pltpu.make_async_copy(zrow.at[pl.ds(0, TAIL)],
                                  agg_sh.at[pl.ds(N - TAIL, TAIL)], semz).wait()
            if with_deg:
                pltpu.make_async_copy(
                    zdeg.at[pl.ds(0, TAIL)],
                    deg_sh.at[pl.ds(N - TAIL, TAIL)], semz).wait()

    ebase = w * EPW

    def start_idx(g, b):
        off = ebase + g * K
        pltpu.async_copy(src_hbm.at[pl.ds(off, K)], sidx[b], semi[b])
        pltpu.async_copy(dst_hbm.at[pl.ds(off, K)], didx[b], semi[b])

    def wait_idx(g, b):
        off = ebase + g * K
        pltpu.make_async_copy(src_hbm.at[pl.ds(off, K)], sidx[b], semi[b]).wait()
        pltpu.make_async_copy(dst_hbm.at[pl.ds(off, K)], didx[b], semi[b]).wait()

    def start_gather(b):
        pltpu.async_copy(h_hbm.at[sidx[b]], rows[b], semg[b])

    def wait_gather(b):
        pltpu.make_async_copy(h_hbm.at[sidx[b]], rows[b], semg[b]).wait()

    def start_scatter(b):
        pltpu.async_copy(rows[b], agg_sh.at[didx[b]], sems_[b], add=True)
        if with_deg:
            pltpu.async_copy(onesb, deg_sh.at[didx[b]], sems_[b], add=True)

    def wait_scatter(b):
        pltpu.make_async_copy(rows[b], agg_sh.at[didx[b]], sems_[b]).wait()
        if with_deg:
            pltpu.make_async_copy(onesb, deg_sh.at[didx[b]], sems_[b]).wait()

    def step(g, b, lo, hi):
        wait_gather(b)
        start_scatter(b)
        if lo - SD >= 0:
            wait_scatter((b + NBUF - SD) % NBUF)
        if hi + IL <= NCHUNK:
            start_idx(g + IL, (b + IL) % NBUF)
        if hi + GL <= NCHUNK:
            wait_idx(g + GL, (b + GL) % NBUF)
            start_gather((b + GL) % NBUF)

    for g in range(IL):
        start_idx(g, g)
    for g in range(GL):
        wait_idx(g, g)
        start_gather(g)
    wait_zero()
    plsc.subcore_barrier()
    for g in range(NBUF):
        step(g, g, g, g + 1)

    STEADY_END = NBUF + ((NCHUNK - IL - NBUF) // NBUF) * NBUF

    def _outer(g2, carry):
        for j in range(NBUF):
            step(g2 * NBUF + j, j, NBUF, STEADY_END)
        return carry
    lax.fori_loop(1, STEADY_END // NBUF, _outer, 0)

    for g in range(STEADY_END, NCHUNK):
        step(g, g % NBUF, g, g + 1)
    for g in range(NCHUNK - SD, NCHUNK):
        wait_scatter(g % NBUF)

    plsc.subcore_barrier()

    pltpu.sync_copy(agg_sh.at[pl.ds(base, AP)], agg_out.at[c, pl.ds(base, AP)])

    @pl.when(s == NS - 1)
    def _():
        pltpu.sync_copy(agg_sh.at[pl.ds(N - TAIL, TAIL)],
                        agg_out.at[c, pl.ds(N - TAIL, TAIL)])

    if with_deg:
        for j in range(AP // ZB):
            pltpu.sync_copy(deg_sh.at[pl.ds(base + j * ZB, ZB)], zdeg)
            pltpu.sync_copy(zdeg, deg_out.at[pl.ds(c * N + base + j * ZB, ZB)])

        @pl.when(s == NS - 1)
        def _():
            pltpu.sync_copy(deg_sh.at[pl.ds(N - TAIL, TAIL)],
                            zdeg.at[pl.ds(0, TAIL)])
            pltpu.sync_copy(zdeg.at[pl.ds(0, TAIL)],
                            deg_out.at[pl.ds(c * N + N - TAIL, TAIL)])


_round1 = pl.kernel(
    functools.partial(_sc_body, True),
    out_type=(jax.ShapeDtypeStruct((NC, N, D), jnp.float32),
              jax.ShapeDtypeStruct((NC * N,), jnp.float32)),
    mesh=_mesh,
    scratch_types=(
        [pltpu.VMEM((K,), jnp.int32)] * NBUF
        + [pltpu.VMEM((K,), jnp.int32)] * NBUF
        + [pltpu.VMEM((K, D), jnp.float32)] * NBUF
        + [
            pltpu.VMEM((K,), jnp.float32),
            pltpu.VMEM((ZB, D), jnp.float32),
            pltpu.VMEM((ZB,), jnp.float32),
            pltpu.VMEM_SHARED((N, D), jnp.float32),
            pltpu.VMEM_SHARED((N,), jnp.float32),
        ]
        + [pltpu.SemaphoreType.DMA] * (3 * NBUF + 1)
    ),
)

_round2 = pl.kernel(
    functools.partial(_sc_body, False),
    out_type=jax.ShapeDtypeStruct((NC, N, D), jnp.float32),
    mesh=_mesh,
    scratch_types=(
        [pltpu.VMEM((K,), jnp.int32)] * NBUF
        + [pltpu.VMEM((K,), jnp.int32)] * NBUF
        + [pltpu.VMEM((K, D), jnp.float32)] * NBUF
        + [
            pltpu.VMEM((ZB, D), jnp.float32),
            pltpu.VMEM_SHARED((N, D), jnp.float32),
        ]
        + [pltpu.SemaphoreType.DMA] * (3 * NBUF + 1)
    ),
)


_RB = 1000


def _combine_body(agg_ref, deg_ref, h_ref, out_ref):
    msk = ((deg_ref[:, 0] + deg_ref[:, 1]) > 0.0)[:, None]
    agg = agg_ref[0] + agg_ref[1]
    out_ref[...] = jnp.where(msk, agg, h_ref[...])


def _final_body(agg_ref, deg_ref, h_ref, w_ref, b_ref, out_ref):
    msk = ((deg_ref[:, 0] + deg_ref[:, 1]) > 0.0)[:, None]
    h2 = jnp.where(msk, agg_ref[0] + agg_ref[1], h_ref[...])
    acc = lax.dot_general(h2, w_ref[...], (((1,), (1,)), ((), ())),
                          preferred_element_type=jnp.float32,
                          precision=lax.Precision.HIGHEST)
    out_ref[...] = acc + b_ref[...]


_combine = pl.pallas_call(
    _combine_body,
    grid=(N // _RB,),
    in_specs=[
        pl.BlockSpec((NC, _RB, D), lambda i: (0, i, 0)),
        pl.BlockSpec((_RB, NC), lambda i: (i, 0)),
        pl.BlockSpec((_RB, D), lambda i: (i, 0)),
    ],
    out_specs=pl.BlockSpec((_RB, D), lambda i: (i, 0)),
    out_shape=jax.ShapeDtypeStruct((N, D), jnp.float32),
)

_final = pl.pallas_call(
    _final_body,
    grid=(N // _RB,),
    in_specs=[
        pl.BlockSpec((NC, _RB, D), lambda i: (0, i, 0)),
        pl.BlockSpec((_RB, NC), lambda i: (i, 0)),
        pl.BlockSpec((_RB, D), lambda i: (i, 0)),
        pl.BlockSpec((D, D), lambda i: (0, 0)),
        pl.BlockSpec((1, D), lambda i: (0, 0)),
    ],
    out_specs=pl.BlockSpec((_RB, D), lambda i: (i, 0)),
    out_shape=jax.ShapeDtypeStruct((N, D), jnp.float32),
)


def kernel(feature, edge_index, W, b):
    src = edge_index[0]
    dst = edge_index[1]
    agg1, degp = _round1(feature, src, dst)
    deg2 = degp.reshape(NC, N).T
    h1 = _combine(agg1, deg2, feature)
    agg2 = _round2(h1, src, dst)
    return _final(agg2, deg2, h1, W, b.reshape(1, D))


# speedup vs baseline: 13.8951x; 1.0271x over previous
import functools

import jax
import jax.numpy as jnp
from jax import lax
from jax.experimental import pallas as pl
from jax.experimental.pallas import tpu as pltpu
from jax.experimental.pallas import tpu_sc as plsc

N = 10000
E = 320000
D = 128

NC = 2
NS = 16
NW = NC * NS
EPW = E // NW
K = 40
NCHUNK = EPW // K
AP = 624
TAIL = 16
ZB = 48

_mesh = plsc.VectorSubcoreMesh(core_axis_name="c", subcore_axis_name="s")


NBUF = 8
SD = 2
IL = 6
GL = 5


def _sc_body(with_deg, *refs):
    if with_deg:
        (h_hbm, src_hbm, dst_hbm, agg_out, deg_out) = refs[:5]
        refs = refs[5:]
        sidx = refs[0:NBUF]; didx = refs[NBUF:2 * NBUF]; rows = refs[2 * NBUF:3 * NBUF]
        (onesb, zrow, zdeg, agg_sh, deg_sh) = refs[3 * NBUF:3 * NBUF + 5]
        sems = refs[3 * NBUF + 5:]
        semi = sems[0:NBUF]; semg = sems[NBUF:2 * NBUF]; sems_ = sems[2 * NBUF:3 * NBUF]
        semz = sems[3 * NBUF]
    else:
        (h_hbm, src_hbm, dst_hbm, agg_out) = refs[:4]
        refs = refs[4:]
        sidx = refs[0:NBUF]; didx = refs[NBUF:2 * NBUF]; rows = refs[2 * NBUF:3 * NBUF]
        (zrow, agg_sh) = refs[3 * NBUF:3 * NBUF + 2]
        sems = refs[3 * NBUF + 2:]
        semi = sems[0:NBUF]; semg = sems[NBUF:2 * NBUF]; sems_ = sems[2 * NBUF:3 * NBUF]
        semz = sems[3 * NBUF]

    c = lax.axis_index("c")
    s = lax.axis_index("s")
    w = c * NS + s
    zeros16 = jnp.zeros((16,), jnp.float32)

    def _zr(i, carry):
        for k8 in range(D // 16):
            zrow[i, pl.ds(k8 * 16, 16)] = zeros16
        return carry
    lax.fori_loop(0, ZB, _zr, 0)
    base = s * AP
    for j in range(AP // ZB):
        pltpu.async_copy(zrow, agg_sh.at[pl.ds(base + j * ZB, ZB)], semz)

    @pl.when(s == NS - 1)
    def _():
        pltpu.async_copy(zrow.at[pl.ds(0, TAIL)],
                         agg_sh.at[pl.ds(N - TAIL, TAIL)], semz)

    if with_deg:
        def _zd(i, carry):
            zdeg[pl.ds(i * 16, 16)] = zeros16
            return carry
        lax.fori_loop(0, ZB // 16, _zd, 0)
        for j in range(AP // ZB):
            pltpu.async_copy(zdeg, deg_sh.at[pl.ds(base + j * ZB, ZB)], semz)

        @pl.when(s == NS - 1)
        def _():
            pltpu.async_copy(zdeg.at[pl.ds(0, TAIL)],
                             deg_sh.at[pl.ds(N - TAIL, TAIL)], semz)

        ones16 = jnp.ones((16,), jnp.float32)
        def _on(i, carry):
            onesb[pl.ds(i * 16, 16)] = ones16
            return carry
        lax.fori_loop(0, K // 16, _on, 0)
        if K % 16:
            onesb[pl.ds(K - 16, 16)] = ones16

    def wait_zero():
        for j in range(AP // ZB):
            pltpu.make_async_copy(
                zrow, agg_sh.at[pl.ds(base + j * ZB, ZB)], semz).wait()
        if with_deg:
            for j in range(AP // ZB):
                pltpu.make_async_copy(
                    zdeg, deg_sh.at[pl.ds(base + j * ZB, ZB)], semz).wait()

        @pl.when(s == NS - 1)
        def _():
            pltpu.make_async_copy(zrow.at[pl.ds(0, TAIL)],
                                  agg_sh.at[pl.ds(N - TAIL, TAIL)], semz).wait()
            if with_deg:
                pltpu.make_async_copy(
                    zdeg.at[pl.ds(0, TAIL)],
                    deg_sh.at[pl.ds(N - TAIL, TAIL)], semz).wait()

    ebase = w * EPW

    def start_idx(g, b):
        off = ebase + g * K
        pltpu.async_copy(src_hbm.at[pl.ds(off, K)], sidx[b], semi[b])
        pltpu.async_copy(dst_hbm.at[pl.ds(off, K)], didx[b], semi[b])

    def wait_idx(g, b):
        off = ebase + g * K
        pltpu.make_async_copy(src_hbm.at[pl.ds(off, K)], sidx[b], semi[b]).wait()
        pltpu.make_async_copy(dst_hbm.at[pl.ds(off, K)], didx[b], semi[b]).wait()

    def start_gather(b):
        pltpu.async_copy(h_hbm.at[sidx[b]], rows[b], semg[b])

    def wait_gather(b):
        pltpu.make_async_copy(h_hbm.at[sidx[b]], rows[b], semg[b]).wait()

    def start_scatter(b):
        pltpu.async_copy(rows[b], agg_sh.at[didx[b]], sems_[b], add=True)
        if with_deg:
            pltpu.async_copy(onesb, deg_sh.at[didx[b]], sems_[b], add=True)

    def wait_scatter(b):
        pltpu.make_async_copy(rows[b], agg_sh.at[didx[b]], sems_[b]).wait()
        if with_deg:
            pltpu.make_async_copy(onesb, deg_sh.at[didx[b]], sems_[b]).wait()

    def step(g, b, lo, hi):
        wait_gather(b)
        start_scatter(b)
        if lo - SD >= 0:
            wait_scatter((b + NBUF - SD) % NBUF)
        if hi + IL <= NCHUNK:
            start_idx(g + IL, (b + IL) % NBUF)
        if hi + GL <= NCHUNK:
            wait_idx(g + GL, (b + GL) % NBUF)
            start_gather((b + GL) % NBUF)

    for g in range(IL):
        start_idx(g, g)
    for g in range(GL):
        wait_idx(g, g)
        start_gather(g)
    wait_zero()
    plsc.subcore_barrier()
    for g in range(NBUF):
        step(g, g, g, g + 1)

    STEADY_END = NBUF + ((NCHUNK - IL - NBUF) // NBUF) * NBUF

    def _outer(g2, carry):
        for j in range(NBUF):
            step(g2 * NBUF + j, j, NBUF, STEADY_END)
        return carry
    lax.fori_loop(1, STEADY_END // NBUF, _outer, 0)

    for g in range(STEADY_END, NCHUNK):
        step(g, g % NBUF, g, g + 1)
    for g in range(NCHUNK - SD, NCHUNK):
        wait_scatter(g % NBUF)

    plsc.subcore_barrier()

    pltpu.sync_copy(agg_sh.at[pl.ds(base, AP)], agg_out.at[c, pl.ds(base, AP)])

    @pl.when(s == NS - 1)
    def _():
        pltpu.sync_copy(agg_sh.at[pl.ds(N - TAIL, TAIL)],
                        agg_out.at[c, pl.ds(N - TAIL, TAIL)])

    if with_deg:
        for j in range(AP // ZB):
            pltpu.sync_copy(deg_sh.at[pl.ds(base + j * ZB, ZB)], zdeg)
            pltpu.sync_copy(zdeg, deg_out.at[pl.ds(c * N + base + j * ZB, ZB)])

        @pl.when(s == NS - 1)
        def _():
            pltpu.sync_copy(deg_sh.at[pl.ds(N - TAIL, TAIL)],
                            zdeg.at[pl.ds(0, TAIL)])
            pltpu.sync_copy(zdeg.at[pl.ds(0, TAIL)],
                            deg_out.at[pl.ds(c * N + N - TAIL, TAIL)])


_round1 = pl.kernel(
    functools.partial(_sc_body, True),
    out_type=(jax.ShapeDtypeStruct((NC, N, D), jnp.float32),
              jax.ShapeDtypeStruct((NC * N,), jnp.float32)),
    mesh=_mesh,
    scratch_types=(
        [pltpu.VMEM((K,), jnp.int32)] * NBUF
        + [pltpu.VMEM((K,), jnp.int32)] * NBUF
        + [pltpu.VMEM((K, D), jnp.float32)] * NBUF
        + [
            pltpu.VMEM((K,), jnp.float32),
            pltpu.VMEM((ZB, D), jnp.float32),
            pltpu.VMEM((ZB,), jnp.float32),
            pltpu.VMEM_SHARED((N, D), jnp.float32),
            pltpu.VMEM_SHARED((N,), jnp.float32),
        ]
        + [pltpu.SemaphoreType.DMA] * (3 * NBUF + 1)
    ),
)

_round2 = pl.kernel(
    functools.partial(_sc_body, False),
    out_type=jax.ShapeDtypeStruct((NC, N, D), jnp.float32),
    mesh=_mesh,
    scratch_types=(
        [pltpu.VMEM((K,), jnp.int32)] * NBUF
        + [pltpu.VMEM((K,), jnp.int32)] * NBUF
        + [pltpu.VMEM((K, D), jnp.float32)] * NBUF
        + [
            pltpu.VMEM((ZB, D), jnp.float32),
            pltpu.VMEM_SHARED((N, D), jnp.float32),
        ]
        + [pltpu.SemaphoreType.DMA] * (3 * NBUF + 1)
    ),
)


_RB = 1000


def _combine_body(agg_ref, deg_ref, h_ref, out_ref):
    msk = ((deg_ref[:, 0] + deg_ref[:, 1]) > 0.0)[:, None]
    agg = agg_ref[0] + agg_ref[1]
    out_ref[...] = jnp.where(msk, agg, h_ref[...])


def _final_body(agg_ref, deg_ref, h_ref, w_ref, b_ref, out_ref):
    msk = ((deg_ref[:, 0] + deg_ref[:, 1]) > 0.0)[:, None]
    h2 = jnp.where(msk, agg_ref[0] + agg_ref[1], h_ref[...])
    acc = lax.dot_general(h2, w_ref[...], (((1,), (1,)), ((), ())),
                          preferred_element_type=jnp.float32,
                          precision=lax.Precision.HIGHEST)
    out_ref[...] = acc + b_ref[...]


_combine = pl.pallas_call(
    _combine_body,
    grid=(N // _RB,),
    in_specs=[
        pl.BlockSpec((NC, _RB, D), lambda i: (0, i, 0)),
        pl.BlockSpec((_RB, NC), lambda i: (i, 0)),
        pl.BlockSpec((_RB, D), lambda i: (i, 0)),
    ],
    out_specs=pl.BlockSpec((_RB, D), lambda i: (i, 0)),
    out_shape=jax.ShapeDtypeStruct((N, D), jnp.float32),
)

_final = pl.pallas_call(
    _final_body,
    grid=(N // _RB,),
    in_specs=[
        pl.BlockSpec((NC, _RB, D), lambda i: (0, i, 0)),
        pl.BlockSpec((_RB, NC), lambda i: (i, 0)),
        pl.BlockSpec((_RB, D), lambda i: (i, 0)),
        pl.BlockSpec((D, D), lambda i: (0, 0)),
        pl.BlockSpec((1, D), lambda i: (0, 0)),
    ],
    out_specs=pl.BlockSpec((_RB, D), lambda i: (i, 0)),
    out_shape=jax.ShapeDtypeStruct((N, D), jnp.float32),
)


def kernel(feature, edge_index, W, b):
    src = edge_index[0]
    dst = edge_index[1]
    agg1, degp = _round1(feature, src, dst)
    deg2 = degp.reshape(NC, N).T
    h1 = _combine(agg1, deg2, feature)
    agg2 = _round2(h1, src, dst)
    return _final(agg2, deg2, h1, W, b.reshape(1, D))
